# baseline (speedup 1.0000x reference)
import functools

import jax
import jax.numpy as jnp
from jax import lax
from jax.experimental import pallas as pl
from jax.experimental.pallas import tpu as pltpu
from jax.experimental.pallas import tpu_sc as plsc

N = 10000
E = 320000
D = 128
H = 8
C = 16
L = 2
BLK = 128
NP_ = 10112
NBLK = NP_ // BLK

NCORE = 2
NSUB = 16
WK = NCORE * NSUB
EPB = 128
BPW = 79
EPAD = WK * EPB * BPW
RPT = NP_ // NSUB
NR2 = NP_ // 2
NRC = 5120
RPC = NRC // NSUB
BPW2 = EPAD // (NSUB * EPB)


def _sc_mesh():
    return plsc.VectorSubcoreMesh(core_axis_name="c", subcore_axis_name="s",
                                  num_cores=NCORE, num_subcores=NSUB)


def _prep_body(x_ref, ntf_ref, ntep_ref, h0_ref):
    oh = (ntf_ref[...] == lax.broadcasted_iota(jnp.int32, (BLK, 8), 1))
    oh = oh.astype(jnp.float32)
    h0_ref[...] = x_ref[...] + jnp.dot(oh, ntep_ref[...],
                                       preferred_element_type=jnp.float32)


def _dense_a_body(h_ref, hist_ref, wl_ref, bl_ref, wr_ref, br_ref, we_ref,
                  ete_ref, attbd_ref, xl_ref, xr_ref, exs_ref, t8_ref):
    h = h_ref[...]
    xl = jnp.dot(h, wl_ref[...], preferred_element_type=jnp.float32) + bl_ref[...]
    xr = jnp.dot(h, wr_ref[...], preferred_element_type=jnp.float32) + br_ref[...]
    hist = hist_ref[0, :, :8] + hist_ref[1, :, :8]
    cnt = jnp.sum(hist, axis=-1, keepdims=True)
    la = jnp.dot(hist / jnp.maximum(cnt, 1.0), ete_ref[...],
                 preferred_element_type=jnp.float32)
    loop_t = jnp.dot(la, we_ref[...], preferred_element_type=jnp.float32)
    z = xl + xr + loop_t
    z = jnp.maximum(z, 0.2 * z)
    alpha = jnp.dot(z, attbd_ref[...], preferred_element_type=jnp.float32)
    exs_ref[...] = jnp.exp(alpha)
    xl_ref[...] = xl
    xr_ref[...] = xr
    t8_ref[...] = jnp.dot(ete_ref[...], we_ref[...],
                          preferred_element_type=jnp.float32)


def _dense_c_body(msg_ref, den_ref, exs_ref, xl_ref, hres_ref, bias_ref,
                  lng_ref, lnb_ref, r8_ref, h_ref):
    accex = den_ref[:, :H]
    exs = exs_ref[...]
    r8 = r8_ref[...]
    msg = msg_ref[...] + jnp.dot(exs, r8,
                                 preferred_element_type=jnp.float32) * xl_ref[...]
    den = jnp.dot(accex + exs, r8, preferred_element_type=jnp.float32)
    out = msg / (den + 1e-16) + bias_ref[...]
    out = jnp.where(out > 0, out, jnp.exp(jnp.minimum(out, 0.0)) - 1.0)
    h2 = out + hres_ref[...]
    mu = jnp.mean(h2, axis=-1, keepdims=True)
    var = jnp.mean((h2 - mu) ** 2, axis=-1, keepdims=True)
    h_ref[...] = (h2 - mu) * lax.rsqrt(var + 1e-5) * lng_ref[...] + lnb_ref[...]


def _final_body(x_ref, h_ref, wo_ref, bo_ref, gb_ref, out_ref):
    y = jnp.dot(h_ref[...], wo_ref[...], preferred_element_type=jnp.float32)
    out_ref[...] = x_ref[...] + gb_ref[...] * (y + bo_ref[...])


def _row_spec():
    return pl.BlockSpec((BLK, D), lambda i: (i, 0))


def _full_spec(shape):
    nd = len(shape)
    return pl.BlockSpec(shape, lambda i: (0,) * nd)


def kernel(x, edge_index, edge_attr, node_types, node_type_emb, edge_type_emb,
           Wl, bl, Wr, br, We, att, bias, ln_g, ln_b, W_out, b_out, gate):
    f32 = jnp.float32
    xp = jnp.zeros((NP_, D), f32).at[:N].set(x)
    ntf = jnp.full((NP_, 8), -1, jnp.int32).at[:N].set(
        jnp.broadcast_to(node_types.astype(jnp.int32)[:, None], (N, 8)))
    ntep = jnp.zeros((8, D), f32).at[:6].set(node_type_emb)
    ete8 = jnp.zeros((8, D), f32).at[:5].set(edge_type_emb)

    src = edge_index[0].astype(jnp.int32)
    dst = edge_index[1].astype(jnp.int32)
    ea = edge_attr.astype(jnp.int32)
    dst_t = jnp.where(src == dst, jnp.int32(N), dst)
    dst_r = jnp.where(src == dst, jnp.int32(NP_), dst)
    srcp = jnp.zeros((EPAD,), jnp.int32).at[:E].set(src)
    dstp = jnp.full((EPAD,), N, jnp.int32).at[:E].set(dst_t)
    dstr = jnp.full((EPAD,), NP_, jnp.int32).at[:E].set(dst_r)
    eap = jnp.zeros((EPAD,), jnp.int32).at[:E].set(ea)
    att_flat = att.reshape(L, H * C)
    ohm_t = jnp.eye(16, dtype=f32)[:H].reshape(H * 16)
    bfl_t = jnp.bitwise_xor(jnp.arange(16, dtype=jnp.int32)[None, :],
                            jnp.array([8, 4, 2, 1], jnp.int32)[:, None]
                            ).reshape(4 * 16)

    heads = lax.broadcasted_iota(jnp.int32, (D,), 0) // C
    r8 = (heads[None, :] == lax.broadcasted_iota(jnp.int32, (H, D), 0)
          ).astype(f32)
    attbd = jnp.transpose(r8) * jnp.reshape(att, (L, H * C))[:, None, :].transpose(0, 2, 1)

    bias_r = bias.reshape(L, 1, H * C)
    bl_r = bl.reshape(L, 1, H * C)
    br_r = br.reshape(L, 1, H * C)
    lng_r = ln_g.reshape(L, 1, D)
    lnb_r = ln_b.reshape(L, 1, D)
    gb = jnp.broadcast_to(gate.astype(f32), (1, D))
    bo_r = b_out.reshape(1, D)

    grid = (NBLK,)

    h0 = pl.pallas_call(
        _prep_body,
        grid=grid,
        in_specs=[_row_spec(), pl.BlockSpec((BLK, 8), lambda i: (i, 0)),
                  _full_spec((8, D))],
        out_specs=_row_spec(),
        out_shape=jax.ShapeDtypeStruct((NP_, D), f32),
    )(xp, ntf, ntep)

    oh_table = jnp.zeros((8, D), f32).at[
        jnp.arange(8), jnp.arange(8)].set(1.0)
    hist2 = _edge_hist(dstp, eap, oh_table)

    h = h0
    for l in range(L):
        xl, xr, exs, t8 = pl.pallas_call(
            _dense_a_body,
            grid=grid,
            in_specs=[_row_spec(),
                      pl.BlockSpec((2, BLK, D), lambda i: (0, i, 0)),
                      _full_spec((D, D)), _full_spec((1, D)),
                      _full_spec((D, D)), _full_spec((1, D)),
                      _full_spec((D, D)), _full_spec((8, D)),
                      _full_spec((D, H))],
            out_specs=[_row_spec(), _row_spec(),
                       pl.BlockSpec((BLK, 8), lambda i: (i, 0)),
                       _full_spec((8, D))],
            out_shape=[jax.ShapeDtypeStruct((NP_, D), f32),
                       jax.ShapeDtypeStruct((NP_, D), f32),
                       jax.ShapeDtypeStruct((NP_, 8), f32),
                       jax.ShapeDtypeStruct((8, D), f32)],
        )(h, hist2, Wl[l], bl_r[l], Wr[l], br_r[l], We[l], ete8, attbd[l])

        den, msg = _edge_pass(srcp, dstp, dstr, eap, xl, xr, t8, att_flat[l],
                              ohm_t, bfl_t)

        h = pl.pallas_call(
            _dense_c_body,
            grid=grid,
            in_specs=[_row_spec(), _row_spec(),
                      pl.BlockSpec((BLK, 8), lambda i: (i, 0)),
                      _row_spec(), _row_spec(), _full_spec((1, D)),
                      _full_spec((1, D)), _full_spec((1, D)),
                      _full_spec((8, D))],
            out_specs=_row_spec(),
            out_shape=jax.ShapeDtypeStruct((NP_, D), f32),
        )(msg, den, exs, xl, h, bias_r[l], lng_r[l], lnb_r[l], r8)

    out = pl.pallas_call(
        _final_body,
        grid=grid,
        in_specs=[_row_spec(), _row_spec(), _full_spec((D, D)),
                  _full_spec((1, D)), _full_spec((1, D))],
        out_specs=_row_spec(),
        out_shape=jax.ShapeDtypeStruct((NP_, D), f32),
    )(xp, h, W_out, bo_r, gb)
    return out[:N]


def _iota16():
    return lax.iota(jnp.int32, 16)


def _hist_body(dst_hbm, kidx_hbm, oh_hbm, out_hbm,
               dst_s, kidx_s, rows_s, zb_s, acc_sh, sem):
    c = lax.axis_index("c")
    s = lax.axis_index("s")
    wid = s * NCORE + c
    for r in range(8):
        for q in range(D // 16):
            zb_s[r, pl.ds(q * 16, 16)] = jnp.zeros((16,), jnp.float32)
    base = s * RPT

    def _z(k, _):
        pltpu.sync_copy(zb_s, acc_sh.at[pl.ds(base + k * 8, 8)])
        return _

    lax.fori_loop(0, RPT // 8, _z, None)
    plsc.subcore_barrier()

    def _blk(b, _):
        e0 = (wid * BPW + b) * EPB
        pltpu.sync_copy(dst_hbm.at[pl.ds(e0, EPB)], dst_s)
        pltpu.sync_copy(kidx_hbm.at[pl.ds(e0, EPB)], kidx_s)
        pltpu.async_copy(oh_hbm.at[kidx_s], rows_s, sem).wait()
        pltpu.sync_copy(rows_s, acc_sh.at[dst_s], add=True)
        return _

    lax.fori_loop(0, BPW, _blk, None)
    plsc.subcore_barrier()

    def _out(k, _):
        pltpu.sync_copy(acc_sh.at[pl.ds(base + k * 8, 8)],
                        out_hbm.at[c, pl.ds(base + k * 8, 8)])
        return _

    lax.fori_loop(0, RPT // 8, _out, None)


def _hist_call():
  return pl.kernel(
    _hist_body,
    out_type=jax.ShapeDtypeStruct((2, NP_, D), jnp.float32),
    mesh=_sc_mesh(),
    scratch_types=[
        pltpu.VMEM((EPB,), jnp.int32),
        pltpu.VMEM((EPB,), jnp.int32),
        pltpu.VMEM((EPB, D), jnp.float32),
        pltpu.VMEM((8, D), jnp.float32),
        pltpu.VMEM_SHARED((NP_, D), jnp.float32),
        pltpu.SemaphoreType.DMA,
    ],
  )


def _edgea_body(src_hbm, dst_hbm, ea_hbm, xl_hbm, xr_hbm, t8_hbm,
                att_hbm, ohm_hbm, bfl_hbm, ex_hbm, src_s, dst_s, ea_s,
                xlr_s, xrr_s, eer_s, att_s, ohm_s, bfl_s, exf_s,
                sem1, sem2, sem3):
    c = lax.axis_index("c")
    s = lax.axis_index("s")
    wid = s * NCORE + c
    pltpu.sync_copy(att_hbm, att_s)
    pltpu.sync_copy(ohm_hbm, ohm_s)
    pltpu.sync_copy(bfl_hbm, bfl_s)

    def _blk(b, _):
        e0 = (wid * BPW + b) * EPB
        pltpu.sync_copy(src_hbm.at[pl.ds(e0, EPB)], src_s)
        pltpu.sync_copy(dst_hbm.at[pl.ds(e0, EPB)], dst_s)
        pltpu.sync_copy(ea_hbm.at[pl.ds(e0, EPB)], ea_s)
        cp1 = pltpu.async_copy(xl_hbm.at[src_s], xlr_s, sem1)
        cp1.wait()
        cp2 = pltpu.async_copy(xr_hbm.at[dst_s], xrr_s, sem2)
        cp3 = pltpu.async_copy(t8_hbm.at[ea_s], eer_s, sem3)
        cp2.wait()
        cp3.wait()

        def _edge(j, _e):
            exv = jnp.zeros((16,), jnp.float32)
            for q in range(H):
                v = (xlr_s[j, pl.ds(q * 16, 16)] + xrr_s[j, pl.ds(q * 16, 16)]
                     + eer_s[j, pl.ds(q * 16, 16)])
                z = jnp.maximum(v, 0.2 * v)
                sq = z * att_s[pl.ds(q * 16, 16)]
                for k in range(4):
                    sq = sq + jnp.take(sq, bfl_s[pl.ds(k * 16, 16)])
                exv = exv + jnp.exp(sq) * ohm_s[pl.ds(q * 16, 16)]
            exf_s[pl.ds(j * 16, 16)] = exv
            return _e

        lax.fori_loop(0, EPB, _edge, None)
        pltpu.sync_copy(exf_s, ex_hbm.at[pl.ds(e0 * 16, EPB * 16)])
        return _

    lax.fori_loop(0, BPW, _blk, None)


def _edgea_call():
  return pl.kernel(
    _edgea_body,
    out_type=jax.ShapeDtypeStruct((EPAD * 16,), jnp.float32),
    mesh=_sc_mesh(),
    scratch_types=[
        pltpu.VMEM((EPB,), jnp.int32),
        pltpu.VMEM((EPB,), jnp.int32),
        pltpu.VMEM((EPB,), jnp.int32),
        pltpu.VMEM((EPB, D), jnp.float32),
        pltpu.VMEM((EPB, D), jnp.float32),
        pltpu.VMEM((EPB, D), jnp.float32),
        pltpu.VMEM((D,), jnp.float32),
        pltpu.VMEM((H * 16,), jnp.float32),
        pltpu.VMEM((4 * 16,), jnp.int32),
        pltpu.VMEM((EPB * 16,), jnp.float32),
        pltpu.SemaphoreType.DMA,
        pltpu.SemaphoreType.DMA,
        pltpu.SemaphoreType.DMA,
    ],
  )


def _route(dst_s, dstl_s, c):
    def _g(g, _):
        d16 = dst_s[pl.ds(g * 16, 16)]
        loc = d16 - c * NR2
        inb = jnp.logical_and(loc >= 0, loc < NR2)
        dstl_s[pl.ds(g * 16, 16)] = jnp.where(
            inb, loc, jnp.full((16,), NR2, jnp.int32))
        return _

    lax.fori_loop(0, EPB // 16, _g, None)


def _zero_acc(zb_s, acc_sh, s):
    for r in range(8):
        for q in range(D // 16):
            zb_s[r, pl.ds(q * 16, 16)] = jnp.zeros((16,), jnp.float32)
    base = s * RPC

    def _z(k, _):
        pltpu.sync_copy(zb_s, acc_sh.at[pl.ds(base + k * 8, 8)])
        return _

    lax.fori_loop(0, RPC // 8, _z, None)
    return base


def _copy_out(acc_sh, out_hbm, base, c):
    def _o(k, _):
        pltpu.sync_copy(acc_sh.at[pl.ds(base + k * 8, 8)],
                        out_hbm.at[c, pl.ds(base + k * 8, 8)])
        return _

    lax.fori_loop(0, RPC // 8, _o, None)


def _edgeb1_body(dst_hbm, ex_hbm, out_hbm, dst_s, dstl_s, exf_s, row_s,
                 zb_s, acc_sh, sem):
    c = lax.axis_index("c")
    s = lax.axis_index("s")
    base = _zero_acc(zb_s, acc_sh, s)

    def _zr(j, _):
        for q in range(D // 16):
            row_s[j, pl.ds(q * 16, 16)] = jnp.zeros((16,), jnp.float32)
        return _

    lax.fori_loop(0, EPB, _zr, None)
    plsc.subcore_barrier()

    def _blk(b, _):
        e0 = (s * BPW2 + b) * EPB
        pltpu.sync_copy(dst_hbm.at[pl.ds(e0, EPB)], dst_s)
        pltpu.sync_copy(ex_hbm.at[pl.ds(e0 * 16, EPB * 16)], exf_s)
        _route(dst_s, dstl_s, c)

        def _edge(j, _e):
            row_s[j, pl.ds(0, 16)] = exf_s[pl.ds(j * 16, 16)]
            return _e

        lax.fori_loop(0, EPB, _edge, None)
        pltpu.sync_copy(row_s, acc_sh.at[dstl_s], add=True)
        return _

    lax.fori_loop(0, BPW2, _blk, None)
    plsc.subcore_barrier()
    _copy_out(acc_sh, out_hbm, base, c)


def _edgeb1_call():
  return pl.kernel(
    _edgeb1_body,
    out_type=jax.ShapeDtypeStruct((2, NRC, D), jnp.float32),
    mesh=_sc_mesh(),
    scratch_types=[
        pltpu.VMEM((EPB,), jnp.int32),
        pltpu.VMEM((EPB,), jnp.int32),
        pltpu.VMEM((EPB * 16,), jnp.float32),
        pltpu.VMEM((EPB, D), jnp.float32),
        pltpu.VMEM((8, D), jnp.float32),
        pltpu.VMEM_SHARED((NRC, D), jnp.float32),
        pltpu.SemaphoreType.DMA,
    ],
  )


def _edgeb2_body(src_hbm, dst_hbm, ex_hbm, xl_hbm, out_hbm, src_s, dst_s,
                 dstl_s, exf_s, xlr_s, msg_s, zb_s, acc_sh, sem1, sem2):
    c = lax.axis_index("c")
    s = lax.axis_index("s")
    base = _zero_acc(zb_s, acc_sh, s)
    plsc.subcore_barrier()

    def _blk(b, _):
        splat = [jnp.full((16,), q, jnp.int32) for q in range(H)]
        e0 = (s * BPW2 + b) * EPB
        pltpu.sync_copy(src_hbm.at[pl.ds(e0, EPB)], src_s)
        pltpu.sync_copy(dst_hbm.at[pl.ds(e0, EPB)], dst_s)
        pltpu.sync_copy(ex_hbm.at[pl.ds(e0 * 16, EPB * 16)], exf_s)
        cp1 = pltpu.async_copy(xl_hbm.at[src_s], xlr_s, sem1)
        _route(dst_s, dstl_s, c)
        cp1.wait()

        def _edge(j, _e):
            exv = exf_s[pl.ds(j * 16, 16)]
            for q in range(H):
                es = jnp.take(exv, splat[q])
                msg_s[j, pl.ds(q * 16, 16)] = xlr_s[j, pl.ds(q * 16, 16)] * es
            return _e

        lax.fori_loop(0, EPB, _edge, None)
        pltpu.sync_copy(msg_s, acc_sh.at[dstl_s], add=True)
        return _

    lax.fori_loop(0, BPW2, _blk, None)
    plsc.subcore_barrier()
    _copy_out(acc_sh, out_hbm, base, c)


def _edgeb2_call():
  return pl.kernel(
    _edgeb2_body,
    out_type=jax.ShapeDtypeStruct((2, NRC, D), jnp.float32),
    mesh=_sc_mesh(),
    scratch_types=[
        pltpu.VMEM((EPB,), jnp.int32),
        pltpu.VMEM((EPB,), jnp.int32),
        pltpu.VMEM((EPB,), jnp.int32),
        pltpu.VMEM((EPB * 16,), jnp.float32),
        pltpu.VMEM((EPB, D), jnp.float32),
        pltpu.VMEM((EPB, D), jnp.float32),
        pltpu.VMEM((8, D), jnp.float32),
        pltpu.VMEM_SHARED((NRC, D), jnp.float32),
        pltpu.SemaphoreType.DMA,
        pltpu.SemaphoreType.DMA,
    ],
  )


def _edge_hist(dstp, eap, oh_table):
    return _hist_call()(dstp, eap, oh_table)


def _edge_pass(srcp, dstp, dstr, eap, xl, xr, t8, att_l, ohm_t, bfl_t):
    exc = _edgea_call()(srcp, dstp, eap, xl, xr, t8, att_l, ohm_t, bfl_t)
    den2 = _edgeb1_call()(dstr, exc)
    msg2 = _edgeb2_call()(srcp, dstr, exc, xl)
    den = jnp.concatenate([den2[0, :NR2], den2[1, :NR2]], axis=0)
    msg = jnp.concatenate([msg2[0, :NR2], msg2[1, :NR2]], axis=0)
    return den, msg


if __name__ == "__main__":
    pass
